# baseline (speedup 1.0000x reference)
import functools

import jax
import jax.numpy as jnp
from jax import lax
from jax.experimental import pallas as pl
from jax.experimental.pallas import tpu as pltpu
from jax.experimental.pallas import tpu_sc as plsc

_N = 10000
_NP = 10240
_C = 128
_T = 12
_E = 160000
_D = _C * _T
_NCH = 16
_DC = _D // _NCH
_CH_PER_SC = _NCH // 2
_KB = 80
_NTILES = 16
_ET = _E // _NTILES
_NBT = 126
_ETP = _NBT * _KB
_NNT = _NP // _NTILES


def _sc_body(xr, srca, dsta, w1a, w2a, zz, out,
             src_v, dst_v, w1_v, w2_v, rows0, rows1, sc0, sc1, acc,
             sg0, sg1, ss0, ss1):
    cid = lax.axis_index("c")
    sid = lax.axis_index("s")
    n0 = sid * _NNT

    def scatter(j, buf, sem):
        pltpu.async_copy(buf, acc.at[dst_v.at[j]], sem, add=True)

    def wait_scatter(buf, sem):
        pltpu.make_async_copy(buf, acc.at[dst_v.at[0]], sem).wait()

    def compute(j, rows, scb):
        for g in range(_KB // 16):
            wv1 = w1_v[j, pl.ds(g * 16, 16)]
            wv2 = w2_v[j, pl.ds(g * 16, 16)]
            for u in range(16):
                e = g * 16 + u
                w1s = wv1[u]
                w2s = wv2[u]
                for r in range(_DC // 32):
                    ab = rows[e, pl.ds(r * 32, 32)]
                    a, b = plsc.unpack(ab, format=plsc.PackFormat.INTERLEAVED)
                    scb[e, pl.ds(r * 32, 32)] = plsc.pack(
                        a * w1s, b * w1s, format=plsc.PackFormat.INTERLEAVED)
                    scb[e, pl.ds(_DC + r * 32, 32)] = plsc.pack(
                        a * w2s, b * w2s, format=plsc.PackFormat.INTERLEAVED)

    pltpu.sync_copy(zz, acc.at[pl.ds(n0, _NNT)])

    def pass_body(k, carry):
        si = k // _CH_PER_SC
        ch = cid * _CH_PER_SC + lax.rem(k, _CH_PER_SC)
        c0 = ch * _DC
        xch = xr.at[pl.ds(ch * _N, _N)]

        def gather(j, buf, sem):
            pltpu.async_copy(xch.at[src_v.at[j]], buf, sem)

        def wait_gather(buf, sem):
            pltpu.make_async_copy(xch.at[src_v.at[0]], buf, sem).wait()

        @pl.when(lax.rem(k, _CH_PER_SC) == 0)
        def _loads():
            pltpu.sync_copy(srca.at[si, sid], src_v)
            pltpu.sync_copy(dsta.at[si, sid], dst_v)
            pltpu.sync_copy(w1a.at[si, sid], w1_v)
            pltpu.sync_copy(w2a.at[si, sid], w2_v)

        plsc.subcore_barrier()
        gather(0, rows0, sg0)

        def pair(i, c1):
            j0 = 2 * i
            j1 = j0 + 1
            wait_gather(rows0, sg0)
            gather(j1, rows1, sg1)

            @pl.when(i > 0)
            def _():
                wait_scatter(sc0, ss0)

            compute(j0, rows0, sc0)
            scatter(j0, sc0, ss0)
            wait_gather(rows1, sg1)

            @pl.when(i < _NBT // 2 - 1)
            def _():
                gather(j0 + 2, rows0, sg0)

            @pl.when(i > 0)
            def _():
                wait_scatter(sc1, ss1)

            compute(j1, rows1, sc1)
            scatter(j1, sc1, ss1)
            return c1

        lax.fori_loop(0, _NBT // 2, pair, 0)
        wait_scatter(sc0, ss0)
        wait_scatter(sc1, ss1)
        plsc.subcore_barrier()
        pltpu.sync_copy(acc.at[pl.ds(n0, _NNT), pl.ds(0, _DC)],
                        out.at[2 * si, pl.ds(n0, _NNT), pl.ds(c0, _DC)])
        pltpu.sync_copy(acc.at[pl.ds(n0, _NNT), pl.ds(_DC, _DC)],
                        out.at[2 * si + 1, pl.ds(n0, _NNT), pl.ds(c0, _DC)])
        pltpu.sync_copy(zz, acc.at[pl.ds(n0, _NNT)])
        return carry

    lax.fori_loop(0, 2 * _CH_PER_SC, pass_body, 0)


def _sc_agg(xr, srca, dsta, w1a, w2a, zz):
    mesh = plsc.VectorSubcoreMesh(core_axis_name="c", subcore_axis_name="s")
    kern = pl.kernel(
        _sc_body,
        out_type=jax.ShapeDtypeStruct((4, _NP, _D), jnp.bfloat16),
        mesh=mesh,
        compiler_params=pltpu.CompilerParams(use_tc_tiling_on_sc=False,
                                             needs_layout_passes=False),
        scratch_types=[
            pltpu.VMEM((_NBT, _KB), jnp.int32),
            pltpu.VMEM((_NBT, _KB), jnp.int32),
            pltpu.VMEM((_NBT, _KB), jnp.float32),
            pltpu.VMEM((_NBT, _KB), jnp.float32),
            pltpu.VMEM((_KB, _DC), jnp.bfloat16),
            pltpu.VMEM((_KB, _DC), jnp.bfloat16),
            pltpu.VMEM((_KB, 2 * _DC), jnp.bfloat16),
            pltpu.VMEM((_KB, 2 * _DC), jnp.bfloat16),
            pltpu.VMEM_SHARED((_NP, 2 * _DC), jnp.bfloat16),
            pltpu.SemaphoreType.DMA,
            pltpu.SemaphoreType.DMA,
            pltpu.SemaphoreType.DMA,
            pltpu.SemaphoreType.DMA,
        ],
    )
    return kern(xr, srca, dsta, w1a, w2a, zz)


def _tc_matmul(aggr, wcat, bias2):
    bn = 480
    bnn = bn // _T

    def body(a_ref, w_ref, b_ref, o_ref):
        acc = jnp.dot(a_ref[0].astype(jnp.float32), w_ref[0],
                      preferred_element_type=jnp.float32)
        for k in range(1, 4):
            acc += jnp.dot(a_ref[k].astype(jnp.float32), w_ref[k],
                           preferred_element_type=jnp.float32)
        acc = acc + b_ref[...]
        o_ref[...] = jnp.transpose(acc.reshape(bnn, _T, _C), (0, 2, 1))

    return pl.pallas_call(
        body,
        grid=(_N // bnn,),
        in_specs=[pl.BlockSpec((4, bn, _C), lambda i: (0, i, 0)),
                  pl.BlockSpec((4, _C, _C), lambda i: (0, 0, 0)),
                  pl.BlockSpec((1, _C), lambda i: (0, 0))],
        out_specs=pl.BlockSpec((bnn, _C, _T), lambda i: (i, 0, 0)),
        out_shape=jax.ShapeDtypeStruct((_N, _C, _T), jnp.float32),
    )(aggr, wcat, bias2)


def _pad_edges(a, fill):
    a3 = a.reshape(a.shape[0], _NTILES, _ET)
    a3 = jnp.pad(a3, ((0, 0), (0, 0), (0, _ETP - _ET)), constant_values=fill)
    return a3.reshape(a.shape[0], _NTILES, _NBT, _KB)


def kernel(x, fwd_edge_index, fwd_w_1, fwd_w_2, bck_edge_index, bck_w_1, bck_w_2,
           W_fwd_1, W_fwd_2, W_bck_1, W_bck_2, bias):
    n, c, t = x.shape
    assert (n, c, t) == (_N, _C, _T) and fwd_edge_index.shape == (2, _E)

    xt2 = jnp.transpose(x, (0, 2, 1)).reshape(_N, _D)
    xr = (xt2.reshape(_N, _NCH, _DC).transpose(1, 0, 2)
          .reshape(_NCH * _N, _DC).astype(jnp.bfloat16))
    srca = _pad_edges(jnp.stack([fwd_edge_index[0], bck_edge_index[0]]), 0)
    dsta = _pad_edges(jnp.stack([fwd_edge_index[1], bck_edge_index[1]]), _N)
    w1a = _pad_edges(jnp.stack([fwd_w_1, bck_w_1]), 0.0)
    w2a = _pad_edges(jnp.stack([fwd_w_2, bck_w_2]), 0.0)
    zz = jnp.zeros((_NNT, 2 * _DC), jnp.bfloat16)

    out4 = _sc_agg(xr, srca, dsta, w1a, w2a, zz)
    agg = out4.reshape(4, _NP * _T, _C)
    wcat = jnp.stack([W_fwd_1, W_fwd_2, W_bck_1, W_bck_2])
    return _tc_matmul(agg, wcat, bias.reshape(1, _C))


# speedup vs baseline: 1.1004x; 1.1004x over previous
import functools

import jax
import jax.numpy as jnp
from jax import lax
from jax.experimental import pallas as pl
from jax.experimental.pallas import tpu as pltpu
from jax.experimental.pallas import tpu_sc as plsc

_N = 10000
_NP = 10240
_C = 128
_T = 12
_E = 160000
_D = _C * _T
_NCH = 16
_DC = _D // _NCH
_CH_PER_SC = _NCH // 2
_KB = 80
_NTILES = 16
_ET = _E // _NTILES
_NBT = 126
_ETP = _NBT * _KB
_NNT = _NP // _NTILES


def _sc_body(xr, srca, dsta, w1a, w2a, zz, out,
             src_v, dst_v, w1_v, w2_v, rows0, rows1, sc0, sc1, acc,
             sg0, sg1, ss0, ss1):
    cid = lax.axis_index("c")
    sid = lax.axis_index("s")
    n0 = sid * _NNT

    def scatter(j, buf, sem):
        pltpu.async_copy(buf, acc.at[dst_v.at[j]], sem, add=True)

    def wait_scatter(buf, sem):
        pltpu.make_async_copy(buf, acc.at[dst_v.at[0]], sem).wait()

    def compute(j, rows, scb):
        for g in range(_KB // 16):
            wv1 = w1_v[j, pl.ds(g * 16, 16)]
            wv2 = w2_v[j, pl.ds(g * 16, 16)]
            for u in range(16):
                e = g * 16 + u
                w1s = wv1[u]
                w2s = wv2[u]
                for r in range(_DC // 32):
                    ab = rows[e, pl.ds(r * 32, 32)]
                    a, b = plsc.unpack(ab, format=plsc.PackFormat.INTERLEAVED)
                    scb[e, pl.ds(r * 32, 32)] = plsc.pack(
                        a * w1s, b * w1s, format=plsc.PackFormat.INTERLEAVED)
                    scb[e, pl.ds(_DC + r * 32, 32)] = plsc.pack(
                        a * w2s, b * w2s, format=plsc.PackFormat.INTERLEAVED)

    pltpu.sync_copy(zz, acc.at[pl.ds(n0, _NNT)])

    def pass_body(k, carry):
        si = k // _CH_PER_SC
        ch = cid * _CH_PER_SC + lax.rem(k, _CH_PER_SC)
        c0 = ch * _DC
        xch = xr.at[pl.ds(ch * _N, _N)]

        def gather(j, buf, sem):
            pltpu.async_copy(xch.at[src_v.at[j]], buf, sem)

        def wait_gather(buf, sem):
            pltpu.make_async_copy(xch.at[src_v.at[0]], buf, sem).wait()

        @pl.when(lax.rem(k, _CH_PER_SC) == 0)
        def _loads():
            pltpu.sync_copy(srca.at[si, sid], src_v)
            pltpu.sync_copy(dsta.at[si, sid], dst_v)
            pltpu.sync_copy(w1a.at[si, sid], w1_v)
            pltpu.sync_copy(w2a.at[si, sid], w2_v)

        plsc.subcore_barrier()
        gather(0, rows0, sg0)

        def pair(i, c1):
            j0 = 2 * i
            j1 = j0 + 1
            wait_gather(rows0, sg0)
            gather(j1, rows1, sg1)

            @pl.when(i > 0)
            def _():
                wait_scatter(sc0, ss0)

            compute(j0, rows0, sc0)
            scatter(j0, sc0, ss0)
            wait_gather(rows1, sg1)

            @pl.when(i < _NBT // 2 - 1)
            def _():
                gather(j0 + 2, rows0, sg0)

            @pl.when(i > 0)
            def _():
                wait_scatter(sc1, ss1)

            compute(j1, rows1, sc1)
            scatter(j1, sc1, ss1)
            return c1

        lax.fori_loop(0, _NBT // 2, pair, 0)
        wait_scatter(sc0, ss0)
        wait_scatter(sc1, ss1)
        plsc.subcore_barrier()
        pltpu.sync_copy(acc.at[pl.ds(n0, _NNT), pl.ds(0, _DC)],
                        out.at[2 * si, pl.ds(n0, _NNT), pl.ds(c0, _DC)])
        pltpu.sync_copy(acc.at[pl.ds(n0, _NNT), pl.ds(_DC, _DC)],
                        out.at[2 * si + 1, pl.ds(n0, _NNT), pl.ds(c0, _DC)])
        pltpu.sync_copy(zz, acc.at[pl.ds(n0, _NNT)])
        return carry

    lax.fori_loop(0, 2 * _CH_PER_SC, pass_body, 0)


def _sc_agg(xr, srca, dsta, w1a, w2a, zz):
    mesh = plsc.VectorSubcoreMesh(core_axis_name="c", subcore_axis_name="s")
    kern = pl.kernel(
        _sc_body,
        out_type=jax.ShapeDtypeStruct((4, _NP, _D), jnp.bfloat16),
        mesh=mesh,
        compiler_params=pltpu.CompilerParams(use_tc_tiling_on_sc=False,
                                             needs_layout_passes=False),
        scratch_types=[
            pltpu.VMEM((_NBT, _KB), jnp.int32),
            pltpu.VMEM((_NBT, _KB), jnp.int32),
            pltpu.VMEM((_NBT, _KB), jnp.float32),
            pltpu.VMEM((_NBT, _KB), jnp.float32),
            pltpu.VMEM((_KB, _DC), jnp.bfloat16),
            pltpu.VMEM((_KB, _DC), jnp.bfloat16),
            pltpu.VMEM((_KB, 2 * _DC), jnp.bfloat16),
            pltpu.VMEM((_KB, 2 * _DC), jnp.bfloat16),
            pltpu.VMEM_SHARED((_NP, 2 * _DC), jnp.bfloat16),
            pltpu.SemaphoreType.DMA,
            pltpu.SemaphoreType.DMA,
            pltpu.SemaphoreType.DMA,
            pltpu.SemaphoreType.DMA,
        ],
    )
    return kern(xr, srca, dsta, w1a, w2a, zz)


def _tc_matmul(aggr, wcat, bias2):
    nt = _N * _T
    bn = 480

    def body(a_ref, w_ref, b_ref, o_ref):
        acc = jnp.dot(a_ref[0].astype(jnp.float32), w_ref[0],
                      preferred_element_type=jnp.float32)
        for k in range(1, 4):
            acc += jnp.dot(a_ref[k].astype(jnp.float32), w_ref[k],
                           preferred_element_type=jnp.float32)
        o_ref[...] = acc + b_ref[...]

    return pl.pallas_call(
        body,
        grid=(nt // bn,),
        in_specs=[pl.BlockSpec((4, bn, _C), lambda i: (0, i, 0)),
                  pl.BlockSpec((4, _C, _C), lambda i: (0, 0, 0)),
                  pl.BlockSpec((1, _C), lambda i: (0, 0))],
        out_specs=pl.BlockSpec((bn, _C), lambda i: (i, 0)),
        out_shape=jax.ShapeDtypeStruct((nt, _C), jnp.float32),
    )(aggr, wcat, bias2)


def _pad_edges(a, fill):
    a3 = a.reshape(a.shape[0], _NTILES, _ET)
    a3 = jnp.pad(a3, ((0, 0), (0, 0), (0, _ETP - _ET)), constant_values=fill)
    return a3.reshape(a.shape[0], _NTILES, _NBT, _KB)


def kernel(x, fwd_edge_index, fwd_w_1, fwd_w_2, bck_edge_index, bck_w_1, bck_w_2,
           W_fwd_1, W_fwd_2, W_bck_1, W_bck_2, bias):
    n, c, t = x.shape
    assert (n, c, t) == (_N, _C, _T) and fwd_edge_index.shape == (2, _E)

    xt2 = jnp.transpose(x, (0, 2, 1)).reshape(_N, _D)
    xr = (xt2.reshape(_N, _NCH, _DC).transpose(1, 0, 2)
          .reshape(_NCH * _N, _DC).astype(jnp.bfloat16))
    srca = _pad_edges(jnp.stack([fwd_edge_index[0], bck_edge_index[0]]), 0)
    dsta = _pad_edges(jnp.stack([fwd_edge_index[1], bck_edge_index[1]]), _N)
    w1a = _pad_edges(jnp.stack([fwd_w_1, bck_w_1]), 0.0)
    w2a = _pad_edges(jnp.stack([fwd_w_2, bck_w_2]), 0.0)
    zz = jnp.zeros((_NNT, 2 * _DC), jnp.bfloat16)

    out4 = _sc_agg(xr, srca, dsta, w1a, w2a, zz)
    agg = out4.reshape(4, _NP * _T, _C)
    wcat = jnp.stack([W_fwd_1, W_fwd_2, W_bck_1, W_bck_2])
    out2 = _tc_matmul(agg, wcat, bias.reshape(1, _C))
    return jnp.transpose(out2.reshape(_N, _T, _C), (0, 2, 1))


# speedup vs baseline: 1.1238x; 1.0213x over previous
import functools

import jax
import jax.numpy as jnp
from jax import lax
from jax.experimental import pallas as pl
from jax.experimental.pallas import tpu as pltpu
from jax.experimental.pallas import tpu_sc as plsc

_N = 10000
_NP = 10240
_C = 128
_T = 12
_E = 160000
_D = _C * _T
_NCH = 16
_DC = _D // _NCH
_CH_PER_SC = _NCH // 2
_KB = 80
_NTILES = 16
_ET = _E // _NTILES
_NBT = 126
_ETP = _NBT * _KB
_NNT = _NP // _NTILES


def _sc_body(xr, srca, dsta, w1a, w2a, zz, out,
             src_v, dst_v, w1_v, w2_v, rows0, rows1, sc0, sc1, acc,
             sg0, sg1, ss0, ss1):
    cid = lax.axis_index("c")
    sid = lax.axis_index("s")
    n0 = sid * _NNT

    def scatter(j, buf, sem):
        pltpu.async_copy(buf, acc.at[dst_v.at[j]], sem, add=True)

    def wait_scatter(buf, sem):
        pltpu.make_async_copy(buf, acc.at[dst_v.at[0]], sem).wait()

    def compute(j, rows, scb):
        for g in range(_KB // 16):
            wv1 = w1_v[j, pl.ds(g * 16, 16)]
            wv2 = w2_v[j, pl.ds(g * 16, 16)]
            for u in range(16):
                e = g * 16 + u
                w1s = wv1[u]
                w2s = wv2[u]
                for r in range(_DC // 32):
                    ab = rows[e, pl.ds(r * 32, 32)]
                    a, b = plsc.unpack(ab, format=plsc.PackFormat.INTERLEAVED)
                    scb[e, pl.ds(r * 32, 32)] = plsc.pack(
                        a * w1s, b * w1s, format=plsc.PackFormat.INTERLEAVED)
                    scb[e, pl.ds(_DC + r * 32, 32)] = plsc.pack(
                        a * w2s, b * w2s, format=plsc.PackFormat.INTERLEAVED)

    pltpu.sync_copy(zz, acc.at[pl.ds(n0, _NNT)])

    def pass_body(k, carry):
        si = k // _CH_PER_SC
        ch = cid * _CH_PER_SC + lax.rem(k, _CH_PER_SC)
        c0 = ch * _DC
        xch = xr.at[pl.ds(ch * _N, _N)]

        def gather(j, buf, sem):
            pltpu.async_copy(xch.at[src_v.at[j]], buf, sem)

        def wait_gather(buf, sem):
            pltpu.make_async_copy(xch.at[src_v.at[0]], buf, sem).wait()

        @pl.when(lax.rem(k, _CH_PER_SC) == 0)
        def _loads():
            pltpu.sync_copy(srca.at[si, sid], src_v)
            pltpu.sync_copy(dsta.at[si, sid], dst_v)
            pltpu.sync_copy(w1a.at[si, sid], w1_v)
            pltpu.sync_copy(w2a.at[si, sid], w2_v)

        plsc.subcore_barrier()
        gather(0, rows0, sg0)

        def pair(i, c1):
            j0 = 2 * i
            j1 = j0 + 1
            wait_gather(rows0, sg0)
            gather(j1, rows1, sg1)

            @pl.when(i > 0)
            def _():
                wait_scatter(sc0, ss0)

            compute(j0, rows0, sc0)
            scatter(j0, sc0, ss0)
            wait_gather(rows1, sg1)

            @pl.when(i < _NBT // 2 - 1)
            def _():
                gather(j0 + 2, rows0, sg0)

            @pl.when(i > 0)
            def _():
                wait_scatter(sc1, ss1)

            compute(j1, rows1, sc1)
            scatter(j1, sc1, ss1)
            return c1

        lax.fori_loop(0, _NBT // 2, pair, 0)
        wait_scatter(sc0, ss0)
        wait_scatter(sc1, ss1)
        plsc.subcore_barrier()
        pltpu.sync_copy(acc.at[pl.ds(n0, _NNT), pl.ds(0, _DC)],
                        out.at[2 * si, pl.ds(n0, _NNT), pl.ds(c0, _DC)])
        pltpu.sync_copy(acc.at[pl.ds(n0, _NNT), pl.ds(_DC, _DC)],
                        out.at[2 * si + 1, pl.ds(n0, _NNT), pl.ds(c0, _DC)])
        pltpu.sync_copy(zz, acc.at[pl.ds(n0, _NNT)])
        return carry

    lax.fori_loop(0, 2 * _CH_PER_SC, pass_body, 0)


def _sc_agg(xr, srca, dsta, w1a, w2a, zz):
    mesh = plsc.VectorSubcoreMesh(core_axis_name="c", subcore_axis_name="s")
    kern = pl.kernel(
        _sc_body,
        out_type=jax.ShapeDtypeStruct((4, _NP, _D), jnp.bfloat16),
        mesh=mesh,
        compiler_params=pltpu.CompilerParams(use_tc_tiling_on_sc=False,
                                             needs_layout_passes=False),
        scratch_types=[
            pltpu.VMEM((_NBT, _KB), jnp.int32),
            pltpu.VMEM((_NBT, _KB), jnp.int32),
            pltpu.VMEM((_NBT, _KB), jnp.float32),
            pltpu.VMEM((_NBT, _KB), jnp.float32),
            pltpu.VMEM((_KB, _DC), jnp.bfloat16),
            pltpu.VMEM((_KB, _DC), jnp.bfloat16),
            pltpu.VMEM((_KB, 2 * _DC), jnp.bfloat16),
            pltpu.VMEM((_KB, 2 * _DC), jnp.bfloat16),
            pltpu.VMEM_SHARED((_NP, 2 * _DC), jnp.bfloat16),
            pltpu.SemaphoreType.DMA,
            pltpu.SemaphoreType.DMA,
            pltpu.SemaphoreType.DMA,
            pltpu.SemaphoreType.DMA,
        ],
    )
    return kern(xr, srca, dsta, w1a, w2a, zz)


def _tc_matmul(aggr, wcat, bias2):
    nt = _N * _T
    bn = 960

    def body(a_ref, w_ref, b_ref, o_ref):
        acc = jnp.dot(a_ref[0].astype(jnp.float32), w_ref[0],
                      preferred_element_type=jnp.float32)
        for k in range(1, 4):
            acc += jnp.dot(a_ref[k].astype(jnp.float32), w_ref[k],
                           preferred_element_type=jnp.float32)
        o_ref[...] = acc + b_ref[...]

    return pl.pallas_call(
        body,
        grid=(nt // bn,),
        in_specs=[pl.BlockSpec((4, bn, _C), lambda i: (0, i, 0)),
                  pl.BlockSpec((4, _C, _C), lambda i: (0, 0, 0)),
                  pl.BlockSpec((1, _C), lambda i: (0, 0))],
        out_specs=pl.BlockSpec((bn, _C), lambda i: (i, 0)),
        out_shape=jax.ShapeDtypeStruct((nt, _C), jnp.float32),
    )(aggr, wcat, bias2)


def _pad_edges(a, fill):
    a3 = a.reshape(a.shape[0], _NTILES, _ET)
    a3 = jnp.pad(a3, ((0, 0), (0, 0), (0, _ETP - _ET)), constant_values=fill)
    return a3.reshape(a.shape[0], _NTILES, _NBT, _KB)


def kernel(x, fwd_edge_index, fwd_w_1, fwd_w_2, bck_edge_index, bck_w_1, bck_w_2,
           W_fwd_1, W_fwd_2, W_bck_1, W_bck_2, bias):
    n, c, t = x.shape
    assert (n, c, t) == (_N, _C, _T) and fwd_edge_index.shape == (2, _E)

    xt2 = jnp.transpose(x, (0, 2, 1)).reshape(_N, _D)
    xr = (xt2.reshape(_N, _NCH, _DC).transpose(1, 0, 2)
          .reshape(_NCH * _N, _DC).astype(jnp.bfloat16))
    srca = _pad_edges(jnp.stack([fwd_edge_index[0], bck_edge_index[0]]), 0)
    dsta = _pad_edges(jnp.stack([fwd_edge_index[1], bck_edge_index[1]]), _N)
    w1a = _pad_edges(jnp.stack([fwd_w_1, bck_w_1]), 0.0)
    w2a = _pad_edges(jnp.stack([fwd_w_2, bck_w_2]), 0.0)
    zz = jnp.zeros((_NNT, 2 * _DC), jnp.bfloat16)

    out4 = _sc_agg(xr, srca, dsta, w1a, w2a, zz)
    agg = out4.reshape(4, _NP * _T, _C)
    wcat = jnp.stack([W_fwd_1, W_fwd_2, W_bck_1, W_bck_2])
    out2 = _tc_matmul(agg, wcat, bias.reshape(1, _C))
    return jnp.transpose(out2.reshape(_N, _T, _C), (0, 2, 1))


# speedup vs baseline: 1.1242x; 1.0004x over previous
import jax
import jax.numpy as jnp
from jax import lax
from jax.experimental import pallas as pl
from jax.experimental.pallas import tpu as pltpu
from jax.experimental.pallas import tpu_sc as plsc

_N = 10000
_NP = 10240
_C = 128
_T = 12
_E = 160000
_D = _C * _T
_NCH = 16
_DC = _D // _NCH
_CH_PER_SC = _NCH // 2
_KB = 80
_NTILES = 16
_ET = _E // _NTILES
_NBT = 126
_ETP = _NBT * _KB
_NNT = _NP // _NTILES


def _sc_body(xr, srca, dsta, w1a, w2a, zz, out,
             src_v, dst_v, w1_v, w2_v, rows0, rows1, sc0, sc1, acc,
             sg0, sg1, ss0, ss1):
    cid = lax.axis_index("c")
    sid = lax.axis_index("s")
    n0 = sid * _NNT

    def scatter(j, buf, sem):
        pltpu.async_copy(buf, acc.at[dst_v.at[j]], sem, add=True)

    def wait_scatter(buf, sem):
        pltpu.make_async_copy(buf, acc.at[dst_v.at[0]], sem).wait()

    def compute(j, rows, scb):
        for g in range(_KB // 16):
            wv1 = w1_v[j, pl.ds(g * 16, 16)]
            wv2 = w2_v[j, pl.ds(g * 16, 16)]
            for u in range(16):
                e = g * 16 + u
                w1s = wv1[u]
                w2s = wv2[u]
                for r in range(_DC // 32):
                    ab = rows[e, pl.ds(r * 32, 32)]
                    a, b = plsc.unpack(ab, format=plsc.PackFormat.INTERLEAVED)
                    scb[e, pl.ds(r * 32, 32)] = plsc.pack(
                        a * w1s, b * w1s, format=plsc.PackFormat.INTERLEAVED)
                    scb[e, pl.ds(_DC + r * 32, 32)] = plsc.pack(
                        a * w2s, b * w2s, format=plsc.PackFormat.INTERLEAVED)

    pltpu.sync_copy(zz, acc.at[pl.ds(n0, _NNT)])

    def pass_body(k, carry):
        si = k // _CH_PER_SC
        ch = cid * _CH_PER_SC + lax.rem(k, _CH_PER_SC)
        c0 = ch * _DC
        xch = xr.at[pl.ds(ch * _N, _N)]

        def gather(j, buf, sem):
            pltpu.async_copy(xch.at[src_v.at[j]], buf, sem)

        def wait_gather(buf, sem):
            pltpu.make_async_copy(xch.at[src_v.at[0]], buf, sem).wait()

        @pl.when(lax.rem(k, _CH_PER_SC) == 0)
        def _loads():
            pltpu.sync_copy(srca.at[si, sid], src_v)
            pltpu.sync_copy(dsta.at[si, sid], dst_v)
            pltpu.sync_copy(w1a.at[si, sid], w1_v)
            pltpu.sync_copy(w2a.at[si, sid], w2_v)

        plsc.subcore_barrier()
        gather(0, rows0, sg0)

        def pair(i, c1):
            j0 = 2 * i
            j1 = j0 + 1
            wait_gather(rows0, sg0)
            gather(j1, rows1, sg1)

            @pl.when(i > 0)
            def _():
                wait_scatter(sc0, ss0)

            compute(j0, rows0, sc0)
            scatter(j0, sc0, ss0)
            wait_gather(rows1, sg1)

            @pl.when(i < _NBT // 2 - 1)
            def _():
                gather(j0 + 2, rows0, sg0)

            @pl.when(i > 0)
            def _():
                wait_scatter(sc1, ss1)

            compute(j1, rows1, sc1)
            scatter(j1, sc1, ss1)
            return c1

        lax.fori_loop(0, _NBT // 2, pair, 0)
        wait_scatter(sc0, ss0)
        wait_scatter(sc1, ss1)
        plsc.subcore_barrier()
        pltpu.sync_copy(acc.at[pl.ds(n0, _NNT), pl.ds(0, _DC)],
                        out.at[2 * si, pl.ds(n0, _NNT), pl.ds(c0, _DC)])
        pltpu.sync_copy(acc.at[pl.ds(n0, _NNT), pl.ds(_DC, _DC)],
                        out.at[2 * si + 1, pl.ds(n0, _NNT), pl.ds(c0, _DC)])
        pltpu.sync_copy(zz, acc.at[pl.ds(n0, _NNT)])
        return carry

    lax.fori_loop(0, 2 * _CH_PER_SC, pass_body, 0)


def _sc_agg(xr, srca, dsta, w1a, w2a, zz):
    mesh = plsc.VectorSubcoreMesh(core_axis_name="c", subcore_axis_name="s")
    kern = pl.kernel(
        _sc_body,
        out_type=jax.ShapeDtypeStruct((4, _NP, _D), jnp.bfloat16),
        mesh=mesh,
        compiler_params=pltpu.CompilerParams(use_tc_tiling_on_sc=False,
                                             needs_layout_passes=False),
        scratch_types=[
            pltpu.VMEM((_NBT, _KB), jnp.int32),
            pltpu.VMEM((_NBT, _KB), jnp.int32),
            pltpu.VMEM((_NBT, _KB), jnp.float32),
            pltpu.VMEM((_NBT, _KB), jnp.float32),
            pltpu.VMEM((_KB, _DC), jnp.bfloat16),
            pltpu.VMEM((_KB, _DC), jnp.bfloat16),
            pltpu.VMEM((_KB, 2 * _DC), jnp.bfloat16),
            pltpu.VMEM((_KB, 2 * _DC), jnp.bfloat16),
            pltpu.VMEM_SHARED((_NP, 2 * _DC), jnp.bfloat16),
            pltpu.SemaphoreType.DMA,
            pltpu.SemaphoreType.DMA,
            pltpu.SemaphoreType.DMA,
            pltpu.SemaphoreType.DMA,
        ],
    )
    return kern(xr, srca, dsta, w1a, w2a, zz)


def _tc_matmul(aggr, wcat, bias2):
    nt = _N * _T
    bn = 960

    def body(a_ref, w_ref, b_ref, o_ref):
        acc = jnp.dot(a_ref[0].astype(jnp.float32), w_ref[0],
                      preferred_element_type=jnp.float32)
        for k in range(1, 4):
            acc += jnp.dot(a_ref[k].astype(jnp.float32), w_ref[k],
                           preferred_element_type=jnp.float32)
        o_ref[...] = acc + b_ref[...]

    return pl.pallas_call(
        body,
        grid=(nt // bn,),
        in_specs=[pl.BlockSpec((4, bn, _C), lambda i: (0, i, 0)),
                  pl.BlockSpec((4, _C, _C), lambda i: (0, 0, 0)),
                  pl.BlockSpec((1, _C), lambda i: (0, 0))],
        out_specs=pl.BlockSpec((bn, _C), lambda i: (i, 0)),
        out_shape=jax.ShapeDtypeStruct((nt, _C), jnp.float32),
    )(aggr, wcat, bias2)


def _pad_edges(a, fill):
    a3 = a.reshape(a.shape[0], _NTILES, _ET)
    a3 = jnp.pad(a3, ((0, 0), (0, 0), (0, _ETP - _ET)), constant_values=fill)
    return a3.reshape(a.shape[0], _NTILES, _NBT, _KB)


def kernel(x, fwd_edge_index, fwd_w_1, fwd_w_2, bck_edge_index, bck_w_1, bck_w_2,
           W_fwd_1, W_fwd_2, W_bck_1, W_bck_2, bias):
    n, c, t = x.shape
    assert (n, c, t) == (_N, _C, _T) and fwd_edge_index.shape == (2, _E)

    xt2 = jnp.transpose(x, (0, 2, 1)).reshape(_N, _D)
    xr = (xt2.reshape(_N, _NCH, _DC).transpose(1, 0, 2)
          .reshape(_NCH * _N, _DC).astype(jnp.bfloat16))
    srca = _pad_edges(jnp.stack([fwd_edge_index[0], bck_edge_index[0]]), 0)
    dsta = _pad_edges(jnp.stack([fwd_edge_index[1], bck_edge_index[1]]), _N)
    w1a = _pad_edges(jnp.stack([fwd_w_1, bck_w_1]), 0.0)
    w2a = _pad_edges(jnp.stack([fwd_w_2, bck_w_2]), 0.0)
    zz = jnp.zeros((_NNT, 2 * _DC), jnp.bfloat16)

    out4 = _sc_agg(xr, srca, dsta, w1a, w2a, zz)
    agg = out4.reshape(4, _NP * _T, _C)
    wcat = jnp.stack([W_fwd_1, W_fwd_2, W_bck_1, W_bck_2])
    out2 = _tc_matmul(agg, wcat, bias.reshape(1, _C))
    return jnp.transpose(out2.reshape(_N, _T, _C), (0, 2, 1))
